# baseline (speedup 1.0000x reference)
import functools

import jax
import jax.numpy as jnp
from jax.experimental import pallas as pl
from jax.experimental.pallas import tpu as pltpu

NN = 50
TK = 20
EMB = 32
G = 8
JP = 64


def _proj_kernel(x_ref, w_ref, bt_ref, xl_ref, xr_ref):
    xlr = jnp.dot(x_ref[...], w_ref[...],
                  preferred_element_type=jnp.float32,
                  precision=jax.lax.Precision.HIGHEST)
    xl_ref[...] = xlr[:, :EMB]
    xr_ref[...] = xlr[:, EMB:] + bt_ref[...]


def _attn_kernel(xl_ref, xrf_ref, delta_ref, onehot_ref, m_ref,
                 val_ref, idx_ref):
    R = G * NN
    u = jnp.concatenate([xl_ref[...], onehot_ref[...]], axis=1)
    w2 = jnp.concatenate([delta_ref[...], xrf_ref[...]], axis=0)
    t = jnp.dot(u, w2, preferred_element_type=jnp.float32,
                  precision=jax.lax.Precision.HIGHEST)
    e = jnp.where(t >= 0, t, 0.2 * t)
    s = jnp.dot(e, m_ref[...], preferred_element_type=jnp.float32,
                  precision=jax.lax.Precision.HIGHEST)
    jj = jax.lax.broadcasted_iota(jnp.int32, (R, JP), 1)
    s = jnp.where(jj >= NN, -jnp.inf, s)
    mx = jnp.max(s, axis=-1, keepdims=True)
    ex = jnp.exp(s - mx)
    denom = jnp.sum(ex, axis=-1, keepdims=True)
    p = ex / (denom + 1e-16)
    ii = jax.lax.broadcasted_iota(jnp.int32, (R, JP), 0) % NN
    p = jnp.where(ii == jj, -1.0, p)
    vals = []
    idxs = []
    for _ in range(TK):
        mv = jnp.max(p, axis=-1)
        am = jnp.argmax(p, axis=-1).astype(jnp.int32)
        vals.append(mv)
        idxs.append(am)
        p = jnp.where(jj == am[:, None], -2.0, p)
    val = jnp.stack(vals, axis=-1)
    idx = jnp.stack(idxs, axis=-1)
    row = jax.lax.broadcasted_iota(jnp.int32, (R, 1), 0)
    base = pl.program_id(0) * R + (row // NN) * NN
    val_ref[...] = val
    idx_ref[...] = idx + base


def kernel(x, edge_index, batch, W_l, b_l, W_r, b_r, att):
    n_total, IN = x.shape
    b = n_total // NN
    grid = b // G
    R = G * NN
    wcat = jnp.concatenate([W_l, W_r], axis=1)
    bt = (b_l + b_r)[None, :]
    xl, xr = pl.pallas_call(
        _proj_kernel,
        grid=(grid,),
        in_specs=[
            pl.BlockSpec((R, IN), lambda i: (i, 0)),
            pl.BlockSpec((IN, 2 * EMB), lambda i: (0, 0)),
            pl.BlockSpec((1, EMB), lambda i: (0, 0)),
        ],
        out_specs=[
            pl.BlockSpec((R, EMB), lambda i: (i, 0)),
            pl.BlockSpec((R, EMB), lambda i: (i, 0)),
        ],
        out_shape=[
            jax.ShapeDtypeStruct((n_total, EMB), jnp.float32),
            jax.ShapeDtypeStruct((n_total, EMB), jnp.float32),
        ],
    )(x, wcat, bt)
    xr_flat = xr.reshape(b, NN * EMB)
    delta = jnp.tile(jnp.eye(EMB, dtype=jnp.float32), (1, NN))
    onehot = (jnp.arange(R)[:, None] // NN
              == jnp.arange(G)[None, :]).astype(jnp.float32)
    m = jnp.concatenate(
        [jnp.kron(jnp.eye(NN, dtype=jnp.float32), att.reshape(EMB, 1)),
         jnp.zeros((NN * EMB, JP - NN), jnp.float32)], axis=1)
    val, idx = pl.pallas_call(
        _attn_kernel,
        grid=(grid,),
        in_specs=[
            pl.BlockSpec((R, EMB), lambda i: (i, 0)),
            pl.BlockSpec((G, NN * EMB), lambda i: (i, 0)),
            pl.BlockSpec((EMB, NN * EMB), lambda i: (0, 0)),
            pl.BlockSpec((R, G), lambda i: (0, 0)),
            pl.BlockSpec((NN * EMB, JP), lambda i: (0, 0)),
        ],
        out_specs=[
            pl.BlockSpec((R, TK), lambda i: (i, 0)),
            pl.BlockSpec((R, TK), lambda i: (i, 0)),
        ],
        out_shape=[
            jax.ShapeDtypeStruct((n_total, TK), jnp.float32),
            jax.ShapeDtypeStruct((n_total, TK), jnp.int32),
        ],
    )(xl, xr_flat, delta, onehot, m)
    attention = val.reshape(-1)
    index_j = idx.reshape(-1)
    offsets = jnp.arange(b, dtype=jnp.int32) * NN
    index_i = (offsets[:, None]
               + jnp.repeat(jnp.arange(NN, dtype=jnp.int32), TK)[None, :]
               ).reshape(-1)
    new_edge_index = jnp.stack([index_i, index_j])
    return (new_edge_index, attention)


# speedup vs baseline: 1.0049x; 1.0049x over previous
import functools

import jax
import jax.numpy as jnp
from jax.experimental import pallas as pl
from jax.experimental.pallas import tpu as pltpu

NN = 50
TK = 20
EMB = 32
G = 8
JP = 64


def _proj_kernel(x_ref, w_ref, bt_ref, xl_ref, xr_ref):
    xlr = jnp.dot(x_ref[...], w_ref[...],
                  preferred_element_type=jnp.float32)
    xl_ref[...] = xlr[:, :EMB]
    xr_ref[...] = xlr[:, EMB:] + bt_ref[...]


def _attn_kernel(xl_ref, xrf_ref, delta_ref, onehot_ref, m_ref,
                 val_ref, idx_ref):
    R = G * NN
    u = jnp.concatenate([xl_ref[...], onehot_ref[...]], axis=1)
    w2 = jnp.concatenate([delta_ref[...], xrf_ref[...]], axis=0)
    t = jnp.dot(u, w2, preferred_element_type=jnp.float32,
                  precision=jax.lax.Precision.HIGHEST)
    e = jnp.where(t >= 0, t, 0.2 * t)
    s = jnp.dot(e, m_ref[...], preferred_element_type=jnp.float32,
                  precision=jax.lax.Precision.HIGHEST)
    jj = jax.lax.broadcasted_iota(jnp.int32, (R, JP), 1)
    s = jnp.where(jj >= NN, -jnp.inf, s)
    mx = jnp.max(s, axis=-1, keepdims=True)
    ex = jnp.exp(s - mx)
    denom = jnp.sum(ex, axis=-1, keepdims=True)
    p = ex / (denom + 1e-16)
    ii = jax.lax.broadcasted_iota(jnp.int32, (R, JP), 0) % NN
    p = jnp.where(ii == jj, -1.0, p)
    vals = []
    idxs = []
    for _ in range(TK):
        mv = jnp.max(p, axis=-1)
        am = jnp.argmax(p, axis=-1).astype(jnp.int32)
        vals.append(mv)
        idxs.append(am)
        p = jnp.where(jj == am[:, None], -2.0, p)
    val = jnp.stack(vals, axis=-1)
    idx = jnp.stack(idxs, axis=-1)
    row = jax.lax.broadcasted_iota(jnp.int32, (R, 1), 0)
    base = pl.program_id(0) * R + (row // NN) * NN
    val_ref[...] = val
    idx_ref[...] = idx + base


def kernel(x, edge_index, batch, W_l, b_l, W_r, b_r, att):
    n_total, IN = x.shape
    b = n_total // NN
    grid = b // G
    R = G * NN
    wcat = jnp.concatenate([W_l, W_r], axis=1)
    bt = (b_l + b_r)[None, :]
    xl, xr = pl.pallas_call(
        _proj_kernel,
        grid=(grid,),
        in_specs=[
            pl.BlockSpec((R, IN), lambda i: (i, 0)),
            pl.BlockSpec((IN, 2 * EMB), lambda i: (0, 0)),
            pl.BlockSpec((1, EMB), lambda i: (0, 0)),
        ],
        out_specs=[
            pl.BlockSpec((R, EMB), lambda i: (i, 0)),
            pl.BlockSpec((R, EMB), lambda i: (i, 0)),
        ],
        out_shape=[
            jax.ShapeDtypeStruct((n_total, EMB), jnp.float32),
            jax.ShapeDtypeStruct((n_total, EMB), jnp.float32),
        ],
    )(x, wcat, bt)
    xr_flat = xr.reshape(b, NN * EMB)
    delta = jnp.tile(jnp.eye(EMB, dtype=jnp.float32), (1, NN))
    onehot = (jnp.arange(R)[:, None] // NN
              == jnp.arange(G)[None, :]).astype(jnp.float32)
    m = jnp.concatenate(
        [jnp.kron(jnp.eye(NN, dtype=jnp.float32), att.reshape(EMB, 1)),
         jnp.zeros((NN * EMB, JP - NN), jnp.float32)], axis=1)
    val, idx = pl.pallas_call(
        _attn_kernel,
        grid=(grid,),
        in_specs=[
            pl.BlockSpec((R, EMB), lambda i: (i, 0)),
            pl.BlockSpec((G, NN * EMB), lambda i: (i, 0)),
            pl.BlockSpec((EMB, NN * EMB), lambda i: (0, 0)),
            pl.BlockSpec((R, G), lambda i: (0, 0)),
            pl.BlockSpec((NN * EMB, JP), lambda i: (0, 0)),
        ],
        out_specs=[
            pl.BlockSpec((R, TK), lambda i: (i, 0)),
            pl.BlockSpec((R, TK), lambda i: (i, 0)),
        ],
        out_shape=[
            jax.ShapeDtypeStruct((n_total, TK), jnp.float32),
            jax.ShapeDtypeStruct((n_total, TK), jnp.int32),
        ],
    )(xl, xr_flat, delta, onehot, m)
    attention = val.reshape(-1)
    index_j = idx.reshape(-1)
    offsets = jnp.arange(b, dtype=jnp.int32) * NN
    index_i = (offsets[:, None]
               + jnp.repeat(jnp.arange(NN, dtype=jnp.int32), TK)[None, :]
               ).reshape(-1)
    new_edge_index = jnp.stack([index_i, index_j])
    return (new_edge_index, attention)


# speedup vs baseline: 1.7493x; 1.7408x over previous
import functools

import jax
import jax.numpy as jnp
from jax.experimental import pallas as pl
from jax.experimental.pallas import tpu as pltpu

NN = 50
TK = 20
EMB = 32
G = 8
JP = 64


def _proj_kernel(x_ref, w_ref, bt_ref, xl_ref, xr_ref):
    xlr = jnp.dot(x_ref[...], w_ref[...],
                  preferred_element_type=jnp.float32)
    xl_ref[...] = xlr[:, :EMB]
    xr_ref[...] = xlr[:, EMB:] + bt_ref[...]


def _attn_kernel(xl_ref, xrf_ref, delta_ref, onehot_ref, m_ref,
                 val_ref, idx_ref):
    R = G * NN
    u = jnp.concatenate([xl_ref[...], onehot_ref[...]], axis=1)
    w2 = jnp.concatenate([delta_ref[...], xrf_ref[...]], axis=0)
    t = jnp.dot(u, w2, preferred_element_type=jnp.float32)
    e = jnp.where(t >= 0, t, 0.2 * t)
    s = jnp.dot(e, m_ref[...], preferred_element_type=jnp.float32)
    jj = jax.lax.broadcasted_iota(jnp.int32, (R, JP), 1)
    s = jnp.where(jj >= NN, -jnp.inf, s)
    mx = jnp.max(s, axis=-1, keepdims=True)
    ex = jnp.exp(s - mx)
    denom = jnp.sum(ex, axis=-1, keepdims=True)
    p = ex / (denom + 1e-16)
    ii = jax.lax.broadcasted_iota(jnp.int32, (R, JP), 0) % NN
    p = jnp.where(ii == jj, -1.0, p)
    vals = []
    idxs = []
    for _ in range(TK):
        mv = jnp.max(p, axis=-1)
        am = jnp.argmax(p, axis=-1).astype(jnp.int32)
        vals.append(mv)
        idxs.append(am)
        p = jnp.where(jj == am[:, None], -2.0, p)
    val = jnp.stack(vals, axis=-1)
    idx = jnp.stack(idxs, axis=-1)
    row = jax.lax.broadcasted_iota(jnp.int32, (R, 1), 0)
    base = pl.program_id(0) * R + (row // NN) * NN
    val_ref[...] = val
    idx_ref[...] = idx + base


def kernel(x, edge_index, batch, W_l, b_l, W_r, b_r, att):
    n_total, IN = x.shape
    b = n_total // NN
    grid = b // G
    R = G * NN
    wcat = jnp.concatenate([W_l, W_r], axis=1)
    bt = (b_l + b_r)[None, :]
    xl, xr = pl.pallas_call(
        _proj_kernel,
        grid=(grid,),
        in_specs=[
            pl.BlockSpec((R, IN), lambda i: (i, 0)),
            pl.BlockSpec((IN, 2 * EMB), lambda i: (0, 0)),
            pl.BlockSpec((1, EMB), lambda i: (0, 0)),
        ],
        out_specs=[
            pl.BlockSpec((R, EMB), lambda i: (i, 0)),
            pl.BlockSpec((R, EMB), lambda i: (i, 0)),
        ],
        out_shape=[
            jax.ShapeDtypeStruct((n_total, EMB), jnp.float32),
            jax.ShapeDtypeStruct((n_total, EMB), jnp.float32),
        ],
    )(x, wcat, bt)
    xr_flat = xr.reshape(b, NN * EMB)
    delta = jnp.tile(jnp.eye(EMB, dtype=jnp.float32), (1, NN))
    onehot = (jnp.arange(R)[:, None] // NN
              == jnp.arange(G)[None, :]).astype(jnp.float32)
    m = jnp.concatenate(
        [jnp.kron(jnp.eye(NN, dtype=jnp.float32), att.reshape(EMB, 1)),
         jnp.zeros((NN * EMB, JP - NN), jnp.float32)], axis=1)
    val, idx = pl.pallas_call(
        _attn_kernel,
        grid=(grid,),
        in_specs=[
            pl.BlockSpec((R, EMB), lambda i: (i, 0)),
            pl.BlockSpec((G, NN * EMB), lambda i: (i, 0)),
            pl.BlockSpec((EMB, NN * EMB), lambda i: (0, 0)),
            pl.BlockSpec((R, G), lambda i: (0, 0)),
            pl.BlockSpec((NN * EMB, JP), lambda i: (0, 0)),
        ],
        out_specs=[
            pl.BlockSpec((R, TK), lambda i: (i, 0)),
            pl.BlockSpec((R, TK), lambda i: (i, 0)),
        ],
        out_shape=[
            jax.ShapeDtypeStruct((n_total, TK), jnp.float32),
            jax.ShapeDtypeStruct((n_total, TK), jnp.int32),
        ],
    )(xl, xr_flat, delta, onehot, m)
    attention = val.reshape(-1)
    index_j = idx.reshape(-1)
    offsets = jnp.arange(b, dtype=jnp.int32) * NN
    index_i = (offsets[:, None]
               + jnp.repeat(jnp.arange(NN, dtype=jnp.int32), TK)[None, :]
               ).reshape(-1)
    new_edge_index = jnp.stack([index_i, index_j])
    return (new_edge_index, attention)


# speedup vs baseline: 1.7777x; 1.0162x over previous
import functools

import jax
import jax.numpy as jnp
from jax.experimental import pallas as pl
from jax.experimental.pallas import tpu as pltpu

NN = 50
TK = 20
EMB = 32
G = 8
JP = 64


def _proj_kernel(x_ref, w_ref, bt_ref, xl_ref, xr_ref):
    xlr = jnp.dot(x_ref[...], w_ref[...],
                  preferred_element_type=jnp.float32)
    xl_ref[...] = xlr[:, :EMB]
    xr_ref[...] = xlr[:, EMB:] + bt_ref[...]


def _attn_kernel(xl_ref, xrf_ref, delta_ref, onehot_ref, m_ref,
                 padm_ref, diagm_ref, val_ref, idx_ref):
    R = G * NN
    u = jnp.concatenate([xl_ref[...], onehot_ref[...]], axis=1)
    w2 = jnp.concatenate([delta_ref[...], xrf_ref[...]], axis=0)
    t = jnp.dot(u, w2, preferred_element_type=jnp.float32)
    e = jnp.maximum(t, 0.2 * t)
    s = jnp.dot(e, m_ref[...], preferred_element_type=jnp.float32)
    sd = jnp.where(padm_ref[...] > 0, -jnp.inf, s)
    ss = jnp.where(diagm_ref[...] > 0, -jnp.inf, sd)
    jj = jax.lax.broadcasted_iota(jnp.int32, (R, JP), 1)
    vals = []
    idxs = []
    for k in range(TK):
        mv = jnp.max(ss, axis=-1)
        am = jnp.argmax(ss, axis=-1).astype(jnp.int32)
        vals.append(mv)
        idxs.append(am)
        if k == 0:
            c = mv[:, None]
        ss = jnp.where(jj == am[:, None], -jnp.inf, ss)
    s20 = jnp.stack(vals, axis=-1)
    idx = jnp.stack(idxs, axis=-1)
    denom = jnp.sum(jnp.exp(sd - c), axis=-1, keepdims=True)
    val = jnp.exp(s20 - c) / (denom + 1e-16)
    row = jax.lax.broadcasted_iota(jnp.int32, (R, 1), 0)
    base = pl.program_id(0) * R + (row // NN) * NN
    val_ref[...] = val
    idx_ref[...] = idx + base


def kernel(x, edge_index, batch, W_l, b_l, W_r, b_r, att):
    n_total, IN = x.shape
    b = n_total // NN
    grid = b // G
    R = G * NN
    wcat = jnp.concatenate([W_l, W_r], axis=1)
    bt = (b_l + b_r)[None, :]
    xl, xr = pl.pallas_call(
        _proj_kernel,
        grid=(grid,),
        in_specs=[
            pl.BlockSpec((R, IN), lambda i: (i, 0)),
            pl.BlockSpec((IN, 2 * EMB), lambda i: (0, 0)),
            pl.BlockSpec((1, EMB), lambda i: (0, 0)),
        ],
        out_specs=[
            pl.BlockSpec((R, EMB), lambda i: (i, 0)),
            pl.BlockSpec((R, EMB), lambda i: (i, 0)),
        ],
        out_shape=[
            jax.ShapeDtypeStruct((n_total, EMB), jnp.float32),
            jax.ShapeDtypeStruct((n_total, EMB), jnp.float32),
        ],
    )(x, wcat, bt)
    xr_flat = xr.reshape(b, NN * EMB)
    delta = jnp.tile(jnp.eye(EMB, dtype=jnp.float32), (1, NN))
    onehot = (jnp.arange(R)[:, None] // NN
              == jnp.arange(G)[None, :]).astype(jnp.float32)
    m = jnp.concatenate(
        [jnp.kron(jnp.eye(NN, dtype=jnp.float32), att.reshape(EMB, 1)),
         jnp.zeros((NN * EMB, JP - NN), jnp.float32)], axis=1)
    lane = jnp.arange(JP, dtype=jnp.int32)[None, :]
    rowi = jnp.arange(R, dtype=jnp.int32)[:, None]
    padm = (lane >= NN).astype(jnp.float32) * jnp.ones((R, 1), jnp.float32)
    diagm = (lane == rowi % NN).astype(jnp.float32)
    val, idx = pl.pallas_call(
        _attn_kernel,
        grid=(grid,),
        in_specs=[
            pl.BlockSpec((R, EMB), lambda i: (i, 0)),
            pl.BlockSpec((G, NN * EMB), lambda i: (i, 0)),
            pl.BlockSpec((EMB, NN * EMB), lambda i: (0, 0)),
            pl.BlockSpec((R, G), lambda i: (0, 0)),
            pl.BlockSpec((NN * EMB, JP), lambda i: (0, 0)),
            pl.BlockSpec((R, JP), lambda i: (0, 0)),
            pl.BlockSpec((R, JP), lambda i: (0, 0)),
        ],
        out_specs=[
            pl.BlockSpec((R, TK), lambda i: (i, 0)),
            pl.BlockSpec((R, TK), lambda i: (i, 0)),
        ],
        out_shape=[
            jax.ShapeDtypeStruct((n_total, TK), jnp.float32),
            jax.ShapeDtypeStruct((n_total, TK), jnp.int32),
        ],
    )(xl, xr_flat, delta, onehot, m, padm, diagm)
    attention = val.reshape(-1)
    index_j = idx.reshape(-1)
    offsets = jnp.arange(b, dtype=jnp.int32) * NN
    index_i = (offsets[:, None]
               + jnp.repeat(jnp.arange(NN, dtype=jnp.int32), TK)[None, :]
               ).reshape(-1)
    new_edge_index = jnp.stack([index_i, index_j])
    return (new_edge_index, attention)


# speedup vs baseline: 1.9540x; 1.0992x over previous
import functools

import jax
import jax.numpy as jnp
from jax.experimental import pallas as pl
from jax.experimental.pallas import tpu as pltpu

NN = 50
TK = 20
EMB = 32
G = 16
JP = 64


def _proj_kernel(x_ref, w_ref, bt_ref, xl_ref, xr_ref):
    xlr = jnp.dot(x_ref[...], w_ref[...],
                  preferred_element_type=jnp.float32)
    xl_ref[...] = xlr[:, :EMB]
    xr_ref[...] = xlr[:, EMB:] + bt_ref[...]


def _attn_kernel(xl_ref, xrf_ref, delta_ref, onehot_ref, m_ref,
                 padm_ref, diagm_ref, val_ref, idx_ref):
    R = G * NN
    u = jnp.concatenate([xl_ref[...], onehot_ref[...]], axis=1)
    w2 = jnp.concatenate([delta_ref[...], xrf_ref[...]], axis=0)
    t = jnp.dot(u, w2, preferred_element_type=jnp.float32)
    e = jnp.maximum(t, 0.2 * t)
    s = jnp.dot(e, m_ref[...], preferred_element_type=jnp.float32)
    sd = jnp.where(padm_ref[...] > 0, -jnp.inf, s)
    ss = jnp.where(diagm_ref[...] > 0, -jnp.inf, sd)
    jj = jax.lax.broadcasted_iota(jnp.int32, (R, JP), 1)
    vals = []
    idxs = []
    for k in range(TK):
        mv = jnp.max(ss, axis=-1)
        am = jnp.argmax(ss, axis=-1).astype(jnp.int32)
        vals.append(mv)
        idxs.append(am)
        if k == 0:
            c = mv[:, None]
        ss = jnp.where(jj == am[:, None], -jnp.inf, ss)
    s20 = jnp.stack(vals, axis=-1)
    idx = jnp.stack(idxs, axis=-1)
    denom = jnp.sum(jnp.exp(sd - c), axis=-1, keepdims=True)
    val = jnp.exp(s20 - c) / (denom + 1e-16)
    row = jax.lax.broadcasted_iota(jnp.int32, (R, 1), 0)
    base = pl.program_id(0) * R + (row // NN) * NN
    val_ref[...] = val
    idx_ref[...] = idx + base


def kernel(x, edge_index, batch, W_l, b_l, W_r, b_r, att):
    n_total, IN = x.shape
    b = n_total // NN
    grid = b // G
    R = G * NN
    wcat = jnp.concatenate([W_l, W_r], axis=1)
    bt = (b_l + b_r)[None, :]
    xl, xr = pl.pallas_call(
        _proj_kernel,
        grid=(grid,),
        in_specs=[
            pl.BlockSpec((R, IN), lambda i: (i, 0)),
            pl.BlockSpec((IN, 2 * EMB), lambda i: (0, 0)),
            pl.BlockSpec((1, EMB), lambda i: (0, 0)),
        ],
        out_specs=[
            pl.BlockSpec((R, EMB), lambda i: (i, 0)),
            pl.BlockSpec((R, EMB), lambda i: (i, 0)),
        ],
        out_shape=[
            jax.ShapeDtypeStruct((n_total, EMB), jnp.float32),
            jax.ShapeDtypeStruct((n_total, EMB), jnp.float32),
        ],
    )(x, wcat, bt)
    xr_flat = xr.reshape(b, NN * EMB)
    delta = jnp.tile(jnp.eye(EMB, dtype=jnp.float32), (1, NN))
    onehot = (jnp.arange(R)[:, None] // NN
              == jnp.arange(G)[None, :]).astype(jnp.float32)
    m = jnp.concatenate(
        [jnp.kron(jnp.eye(NN, dtype=jnp.float32), att.reshape(EMB, 1)),
         jnp.zeros((NN * EMB, JP - NN), jnp.float32)], axis=1)
    lane = jnp.arange(JP, dtype=jnp.int32)[None, :]
    rowi = jnp.arange(R, dtype=jnp.int32)[:, None]
    padm = (lane >= NN).astype(jnp.float32) * jnp.ones((R, 1), jnp.float32)
    diagm = (lane == rowi % NN).astype(jnp.float32)
    val, idx = pl.pallas_call(
        _attn_kernel,
        grid=(grid,),
        in_specs=[
            pl.BlockSpec((R, EMB), lambda i: (i, 0)),
            pl.BlockSpec((G, NN * EMB), lambda i: (i, 0)),
            pl.BlockSpec((EMB, NN * EMB), lambda i: (0, 0)),
            pl.BlockSpec((R, G), lambda i: (0, 0)),
            pl.BlockSpec((NN * EMB, JP), lambda i: (0, 0)),
            pl.BlockSpec((R, JP), lambda i: (0, 0)),
            pl.BlockSpec((R, JP), lambda i: (0, 0)),
        ],
        out_specs=[
            pl.BlockSpec((R, TK), lambda i: (i, 0)),
            pl.BlockSpec((R, TK), lambda i: (i, 0)),
        ],
        out_shape=[
            jax.ShapeDtypeStruct((n_total, TK), jnp.float32),
            jax.ShapeDtypeStruct((n_total, TK), jnp.int32),
        ],
    )(xl, xr_flat, delta, onehot, m, padm, diagm)
    attention = val.reshape(-1)
    index_j = idx.reshape(-1)
    offsets = jnp.arange(b, dtype=jnp.int32) * NN
    index_i = (offsets[:, None]
               + jnp.repeat(jnp.arange(NN, dtype=jnp.int32), TK)[None, :]
               ).reshape(-1)
    new_edge_index = jnp.stack([index_i, index_j])
    return (new_edge_index, attention)


# speedup vs baseline: 2.0192x; 1.0334x over previous
import functools

import jax
import jax.numpy as jnp
from jax.experimental import pallas as pl
from jax.experimental.pallas import tpu as pltpu

NN = 50
TK = 20
EMB = 32
G = 32
JP = 64


def _proj_kernel(x_ref, w_ref, bt_ref, xl_ref, xr_ref):
    xlr = jnp.dot(x_ref[...], w_ref[...],
                  preferred_element_type=jnp.float32)
    xl_ref[...] = xlr[:, :EMB]
    xr_ref[...] = xlr[:, EMB:] + bt_ref[...]


def _attn_kernel(xl_ref, xrf_ref, delta_ref, onehot_ref, m_ref,
                 padm_ref, diagm_ref, val_ref, idx_ref):
    R = G * NN
    u = jnp.concatenate([xl_ref[...], onehot_ref[...]], axis=1)
    w2 = jnp.concatenate([delta_ref[...], xrf_ref[...]], axis=0)
    t = jnp.dot(u, w2, preferred_element_type=jnp.float32)
    e = jnp.maximum(t, 0.2 * t)
    s = jnp.dot(e, m_ref[...], preferred_element_type=jnp.float32)
    sd = jnp.where(padm_ref[...] > 0, -jnp.inf, s)
    ss = jnp.where(diagm_ref[...] > 0, -jnp.inf, sd)
    jj = jax.lax.broadcasted_iota(jnp.int32, (R, JP), 1)
    vals = []
    idxs = []
    for k in range(TK):
        mv = jnp.max(ss, axis=-1)
        am = jnp.argmax(ss, axis=-1).astype(jnp.int32)
        vals.append(mv)
        idxs.append(am)
        if k == 0:
            c = mv[:, None]
        ss = jnp.where(jj == am[:, None], -jnp.inf, ss)
    s20 = jnp.stack(vals, axis=-1)
    idx = jnp.stack(idxs, axis=-1)
    denom = jnp.sum(jnp.exp(sd - c), axis=-1, keepdims=True)
    val = jnp.exp(s20 - c) / (denom + 1e-16)
    row = jax.lax.broadcasted_iota(jnp.int32, (R, 1), 0)
    base = pl.program_id(0) * R + (row // NN) * NN
    val_ref[...] = val
    idx_ref[...] = idx + base


def kernel(x, edge_index, batch, W_l, b_l, W_r, b_r, att):
    n_total, IN = x.shape
    b = n_total // NN
    grid = b // G
    R = G * NN
    wcat = jnp.concatenate([W_l, W_r], axis=1)
    bt = (b_l + b_r)[None, :]
    xl, xr = pl.pallas_call(
        _proj_kernel,
        grid=(grid,),
        in_specs=[
            pl.BlockSpec((R, IN), lambda i: (i, 0)),
            pl.BlockSpec((IN, 2 * EMB), lambda i: (0, 0)),
            pl.BlockSpec((1, EMB), lambda i: (0, 0)),
        ],
        out_specs=[
            pl.BlockSpec((R, EMB), lambda i: (i, 0)),
            pl.BlockSpec((R, EMB), lambda i: (i, 0)),
        ],
        out_shape=[
            jax.ShapeDtypeStruct((n_total, EMB), jnp.float32),
            jax.ShapeDtypeStruct((n_total, EMB), jnp.float32),
        ],
    )(x, wcat, bt)
    xr_flat = xr.reshape(b, NN * EMB)
    delta = jnp.tile(jnp.eye(EMB, dtype=jnp.float32), (1, NN))
    onehot = (jnp.arange(R)[:, None] // NN
              == jnp.arange(G)[None, :]).astype(jnp.float32)
    m = jnp.concatenate(
        [jnp.kron(jnp.eye(NN, dtype=jnp.float32), att.reshape(EMB, 1)),
         jnp.zeros((NN * EMB, JP - NN), jnp.float32)], axis=1)
    lane = jnp.arange(JP, dtype=jnp.int32)[None, :]
    rowi = jnp.arange(R, dtype=jnp.int32)[:, None]
    padm = (lane >= NN).astype(jnp.float32) * jnp.ones((R, 1), jnp.float32)
    diagm = (lane == rowi % NN).astype(jnp.float32)
    val, idx = pl.pallas_call(
        _attn_kernel,
        grid=(grid,),
        in_specs=[
            pl.BlockSpec((R, EMB), lambda i: (i, 0)),
            pl.BlockSpec((G, NN * EMB), lambda i: (i, 0)),
            pl.BlockSpec((EMB, NN * EMB), lambda i: (0, 0)),
            pl.BlockSpec((R, G), lambda i: (0, 0)),
            pl.BlockSpec((NN * EMB, JP), lambda i: (0, 0)),
            pl.BlockSpec((R, JP), lambda i: (0, 0)),
            pl.BlockSpec((R, JP), lambda i: (0, 0)),
        ],
        out_specs=[
            pl.BlockSpec((R, TK), lambda i: (i, 0)),
            pl.BlockSpec((R, TK), lambda i: (i, 0)),
        ],
        out_shape=[
            jax.ShapeDtypeStruct((n_total, TK), jnp.float32),
            jax.ShapeDtypeStruct((n_total, TK), jnp.int32),
        ],
    )(xl, xr_flat, delta, onehot, m, padm, diagm)
    attention = val.reshape(-1)
    index_j = idx.reshape(-1)
    offsets = jnp.arange(b, dtype=jnp.int32) * NN
    index_i = (offsets[:, None]
               + jnp.repeat(jnp.arange(NN, dtype=jnp.int32), TK)[None, :]
               ).reshape(-1)
    new_edge_index = jnp.stack([index_i, index_j])
    return (new_edge_index, attention)
